# baseline (speedup 1.0000x reference)
import functools

import jax
import jax.numpy as jnp
from jax import lax
from jax.experimental import pallas as pl
from jax.experimental.pallas import tpu as pltpu
from jax.experimental.pallas import tpu_sc as plsc

N = 10000
E = 160000
D = 256

NC, NS = 2, 16
NW = NC * NS

BN = 400
GA = 2 * (N // BN)

EPT = 5120
EPAD = NW * EPT
GB = 64
NB_B = EPT // GB

BE = 2048

CHUNK = 320
NPAD = NW * CHUNK
CD = 4000
NCH = E // CD
GD = 64

NEG = -3.0e38
NEG_THRESH = -1.0e37


def _uv_body(x_ref, w1_ref, b1_ref, uv_ref):
    p = pl.program_id(0) // (GA // 2)
    w1a = w1_ref[:D, :]
    w1b = w1_ref[D:, :]
    w = jnp.where(p == 0, w1a - w1b, w1b)
    b = jnp.where(p == 0, b1_ref[...], jnp.zeros_like(b1_ref[...]))
    uv_ref[...] = jnp.dot(x_ref[...], w, preferred_element_type=jnp.float32) + b


def _mlp2_body(m_ref, w2_ref, b2_ref, t_ref):
    t_ref[...] = (
        jnp.dot(m_ref[...], w2_ref[...], preferred_element_type=jnp.float32)
        + b2_ref[...]
    )


def _gather_body(ids_hbm, uv_hbm, msg_hbm, ids_v, gbuf, ob, sem_g, sem_w):
    wid = lax.axis_index("s") * NC + lax.axis_index("c")
    ibase = wid * (2 * EPT)
    ebase = wid * EPT
    pltpu.sync_copy(ids_hbm.at[pl.ds(ibase, 2 * EPT)], ids_v)

    @pl.loop(0, NB_B)
    def _batch(j):
        idx = ids_v.at[pl.ds(j * (2 * GB), 2 * GB)]
        pltpu.async_copy(uv_hbm.at[idx], gbuf, sem_g).wait()

        @pl.loop(0, GB)
        def _edge(jj):
            for k in range(D // 16):
                sl = pl.ds(k * 16, 16)
                u = gbuf[jj, sl]
                v = gbuf[GB + jj, sl]
                ob[jj, sl] = jnp.maximum(u + v, 0.0)

        pltpu.async_copy(ob, msg_hbm.at[pl.ds(ebase + j * GB, GB)], sem_w).wait()


def _scatter_body(row_hbm, t_hbm, out_hbm, rowbuf, eid_buf, nid_buf, acc, grows,
                  sem_g, sem_w):
    wid = lax.axis_index("s") * NC + lax.axis_index("c")
    lo = wid * CHUNK

    @pl.loop(0, CHUNK + 1)
    def _init(r):
        for k in range(D // 16):
            acc[r, pl.ds(k * 16, 16)] = jnp.full((16,), NEG, jnp.float32)

    @pl.loop(0, NCH)
    def _chunk(c):
        pltpu.sync_copy(row_hbm.at[pl.ds(c * CD, CD)], rowbuf)

        @pl.loop(0, CD // 16, init_carry=jnp.int32(0))
        def _filter(g, off):
            rows = rowbuf[pl.ds(g * 16, 16)]
            m = (rows >= lo) & (rows < lo + CHUNK)
            eids = lax.iota(jnp.int32, 16) + (c * CD + g * 16)
            pos = plsc.cumsum(m.astype(jnp.int32))
            idxv = off + pos - 1
            plsc.store_scatter(eid_buf, [idxv], eids, mask=m)
            plsc.store_scatter(nid_buf, [idxv], rows - lo, mask=m)
            return off + pos[15]

        m_cnt = _filter
        sent_n = jnp.full((16,), CHUNK, jnp.int32)
        sent_e = jnp.zeros((16,), jnp.int32)
        for pad in range(GD // 16):
            nid_buf[pl.ds(m_cnt + pad * 16, 16)] = sent_n
            eid_buf[pl.ds(m_cnt + pad * 16, 16)] = sent_e
        nb = (m_cnt + (GD - 1)) >> 6

        @pl.loop(0, nb)
        def _batch(j):
            idx = eid_buf.at[pl.ds(j * GD, GD)]
            pltpu.async_copy(t_hbm.at[idx], grows, sem_g).wait()

            @pl.loop(0, GD // 16)
            def _group(gg):
                nids = nid_buf[pl.ds(j * GD + gg * 16, 16)]
                for jj in range(16):
                    nid = nids[jj]
                    erow = gg * 16 + jj
                    for k in range(D // 16):
                        sl = pl.ds(k * 16, 16)
                        acc[nid, sl] = jnp.maximum(acc[nid, sl], grows[erow, sl])

    @pl.loop(0, CHUNK)
    def _fin(r):
        for k in range(D // 16):
            sl = pl.ds(k * 16, 16)
            a = acc[r, sl]
            acc[r, sl] = jnp.where(a > NEG_THRESH, a, 0.0)

    pltpu.async_copy(acc.at[pl.ds(0, CHUNK)], out_hbm.at[pl.ds(lo, CHUNK)],
                     sem_w).wait()


def kernel(x, edge_index, W1, b1, W2, b2):
    row = edge_index[0].astype(jnp.int32)
    col = edge_index[1].astype(jnp.int32)
    b1r = b1.reshape(1, D).astype(jnp.float32)
    b2r = b2.reshape(1, D).astype(jnp.float32)

    uv = pl.pallas_call(
        _uv_body,
        grid=(GA,),
        in_specs=[
            pl.BlockSpec((BN, D), lambda i: (lax.rem(i, GA // 2), 0)),
            pl.BlockSpec((2 * D, D), lambda i: (0, 0)),
            pl.BlockSpec((1, D), lambda i: (0, 0)),
        ],
        out_specs=pl.BlockSpec((BN, D), lambda i: (i, 0)),
        out_shape=jax.ShapeDtypeStruct((2 * N, D), jnp.float32),
    )(x, W1, b1r)

    pad = jnp.zeros((EPAD - E,), jnp.int32)
    row_p = jnp.concatenate([row, pad]).reshape(-1, GB)
    col_p = jnp.concatenate([col, pad]).reshape(-1, GB) + N
    ids2 = jnp.concatenate([row_p, col_p], axis=1).reshape(-1)

    mesh = plsc.VectorSubcoreMesh(
        core_axis_name="c", subcore_axis_name="s", num_cores=NC, num_subcores=NS
    )

    msg = pl.kernel(
        _gather_body,
        out_type=jax.ShapeDtypeStruct((EPAD, D), jnp.float32),
        mesh=mesh,
        compiler_params=pltpu.CompilerParams(needs_layout_passes=False),
        scratch_types=[
            pltpu.VMEM((2 * EPT,), jnp.int32),
            pltpu.VMEM((2 * GB, D), jnp.float32),
            pltpu.VMEM((GB, D), jnp.float32),
            pltpu.SemaphoreType.DMA,
            pltpu.SemaphoreType.DMA,
        ],
    )(ids2, uv)

    t = pl.pallas_call(
        _mlp2_body,
        grid=(EPAD // BE,),
        in_specs=[
            pl.BlockSpec((BE, D), lambda i: (i, 0)),
            pl.BlockSpec((D, D), lambda i: (0, 0)),
            pl.BlockSpec((1, D), lambda i: (0, 0)),
        ],
        out_specs=pl.BlockSpec((BE, D), lambda i: (i, 0)),
        out_shape=jax.ShapeDtypeStruct((EPAD, D), jnp.float32),
    )(msg, W2, b2r)

    out_pad = pl.kernel(
        _scatter_body,
        out_type=jax.ShapeDtypeStruct((NPAD, D), jnp.float32),
        mesh=mesh,
        compiler_params=pltpu.CompilerParams(needs_layout_passes=False),
        scratch_types=[
            pltpu.VMEM((CD,), jnp.int32),
            pltpu.VMEM((CD + GD,), jnp.int32),
            pltpu.VMEM((CD + GD,), jnp.int32),
            pltpu.VMEM((CHUNK + 1, D), jnp.float32),
            pltpu.VMEM((GD, D), jnp.float32),
            pltpu.SemaphoreType.DMA,
            pltpu.SemaphoreType.DMA,
        ],
    )(row, t)

    return out_pad[:N]
